# baseline (speedup 1.0000x reference)
import functools

import jax
import jax.numpy as jnp
from jax.experimental import pallas as pl
from jax.experimental.pallas import tpu as pltpu

Q = 4
K = 1024
D = 256
TILE = 512


def _rownorm(m):
    sq = m * m
    a = sq[:, :128] + sq[:, 128:]
    at = jnp.swapaxes(a, 0, 1)
    s = at[0:8]
    for j in range(1, 16):
        s = s + at[8 * j:8 * (j + 1)]
    s = s[0:4] + s[4:8]
    s = s[0:2] + s[2:4]
    s = s[0:1] + s[1:2]
    return jnp.swapaxes(s, 0, 1)


def _rvq_body(x_ref, cbt_ref, c123_ref,
              qout_ref, i0_ref, i1_ref, i2_ref, i3_ref, stats_ref,
              counts_ref, commit_ref, cn_ref):
    t = pl.program_id(0)
    nt = pl.num_programs(0)
    bn = nt * TILE

    @pl.when(t == 0)
    def _init():
        counts_ref[...] = jnp.zeros_like(counts_ref)
        commit_ref[...] = jnp.zeros_like(commit_ref)
        stats_ref[...] = jnp.zeros_like(stats_ref)
        for q in range(Q):
            et = cbt_ref[q]
            cn_ref[q:q + 1, :] = jnp.sum(et * et, axis=0, keepdims=True)

    idx_refs = (i0_ref, i1_ref, i2_ref, i3_ref)
    lane_iota = jax.lax.broadcasted_iota(jnp.int32, (TILE, K), 1)
    RB = 128
    iota_rb = jax.lax.broadcasted_iota(jnp.int32, (RB, 128), 1)
    ones8 = jnp.ones((8, TILE), jnp.bfloat16)

    r = x_ref[...]
    qout = jnp.zeros_like(r)
    for q in range(Q):
        rn = _rownorm(r)
        scores = jax.lax.dot_general(
            r, cbt_ref[q], (((1,), (0,)), ((), ())),
            preferred_element_type=jnp.float32)
        idx_parts = []
        for rb in range(TILE // RB):
            rsl = slice(RB * rb, RB * (rb + 1))
            rnb = rn[rsl]
            run_val = (rnb - 2.0 * scores[rsl, 0:128]
                       + cn_ref[q:q + 1, 0:128])
            run_idx = iota_rb
            for c in range(1, K // 128):
                dv = (rnb - 2.0 * scores[rsl, 128 * c:128 * (c + 1)]
                      + cn_ref[q:q + 1, 128 * c:128 * (c + 1)])
                upd = dv < run_val
                run_val = jnp.where(upd, dv, run_val)
                run_idx = jnp.where(upd, iota_rb + 128 * c, run_idx)
            minv = jnp.min(run_val, axis=-1, keepdims=True)
            idx_parts.append(jnp.min(
                jnp.where(run_val == minv, run_idx, K), axis=-1,
                keepdims=True))
        idx2 = jnp.concatenate(idx_parts, axis=0)
        idx_refs[q][...] = idx2
        oh = (lane_iota == idx2).astype(jnp.bfloat16)
        qh3 = jax.lax.dot_general(
            oh, c123_ref[q], (((1,), (0,)), ((), ())),
            preferred_element_type=jnp.float32)
        qh = (qh3[:, :D] + qh3[:, D:2 * D]) + qh3[:, 2 * D:]
        diff = qh - r
        qz = r + diff
        commit_ref[q:q + 1, :] = commit_ref[q:q + 1, :] + jnp.sum(
            diff * diff, axis=0, keepdims=True)
        cnt8 = jax.lax.dot_general(
            ones8, oh, (((1,), (0,)), ((), ())),
            preferred_element_type=jnp.float32)
        counts_ref[q:q + 1, :] = counts_ref[q:q + 1, :] + cnt8[0:1, :]
        r = r - qz
        qout = qout + qz
    qout_ref[...] = qout

    @pl.when(t == nt - 1)
    def _epilogue():
        rows = jax.lax.broadcasted_iota(jnp.int32, (8, 128), 0)
        lanes = jax.lax.broadcasted_iota(jnp.int32, (8, 128), 1)
        stats = jnp.zeros((8, 128), jnp.float32)
        for q in range(Q):
            avg = counts_ref[q:q + 1, :] / float(bn)
            divq = jnp.sum(avg * jnp.log(avg + 1e-10))
            comq = jnp.sum(commit_ref[q:q + 1, :]) / float(bn * D)
            stats = (stats
                     + jnp.where((rows == 0) & (lanes == q), comq, 0.0)
                     + jnp.where((rows == 1) & (lanes == q), divq, 0.0))
        stats_ref[...] = stats


@functools.partial(jax.jit, static_argnames=("interpret",))
def kernel(x, codebooks, interpret=False):
    B, N, _ = x.shape
    bn = B * N
    nt = bn // TILE
    x2 = x.reshape(bn, D)
    cbt = jnp.swapaxes(codebooks, 1, 2)
    mask = jnp.int32(-65536)
    v = jax.lax.bitcast_convert_type(codebooks, jnp.int32)
    h1 = jax.lax.bitcast_convert_type(v & mask, jnp.float32)
    r1 = codebooks - h1
    w = jax.lax.bitcast_convert_type(r1, jnp.int32)
    h2 = jax.lax.bitcast_convert_type(w & mask, jnp.float32)
    r2 = r1 - h2
    c123 = jnp.concatenate([h1.astype(jnp.bfloat16),
                            h2.astype(jnp.bfloat16),
                            r2.astype(jnp.bfloat16)], axis=2)

    outs = pl.pallas_call(
        _rvq_body,
        grid=(nt,),
        in_specs=[
            pl.BlockSpec((TILE, D), lambda t: (t, 0)),
            pl.BlockSpec((Q, D, K), lambda t: (0, 0, 0)),
            pl.BlockSpec((Q, K, 3 * D), lambda t: (0, 0, 0)),
        ],
        out_specs=[
            pl.BlockSpec((TILE, D), lambda t: (t, 0)),
            pl.BlockSpec((TILE, 1), lambda t: (t, 0)),
            pl.BlockSpec((TILE, 1), lambda t: (t, 0)),
            pl.BlockSpec((TILE, 1), lambda t: (t, 0)),
            pl.BlockSpec((TILE, 1), lambda t: (t, 0)),
            pl.BlockSpec((8, 128), lambda t: (0, 0)),
        ],
        out_shape=[
            jax.ShapeDtypeStruct((bn, D), jnp.float32),
            jax.ShapeDtypeStruct((bn, 1), jnp.int32),
            jax.ShapeDtypeStruct((bn, 1), jnp.int32),
            jax.ShapeDtypeStruct((bn, 1), jnp.int32),
            jax.ShapeDtypeStruct((bn, 1), jnp.int32),
            jax.ShapeDtypeStruct((8, 128), jnp.float32),
        ],
        scratch_shapes=[
            pltpu.VMEM((8, K), jnp.float32),
            pltpu.VMEM((8, D), jnp.float32),
            pltpu.VMEM((8, K), jnp.float32),
        ],
        compiler_params=pltpu.CompilerParams(
            dimension_semantics=("arbitrary",)),
        interpret=interpret,
    )(x2, cbt, c123)

    qout, i0, i1, i2, i3, stats = outs
    quantized = qout.reshape(B, N, D)
    all_indices = jnp.concatenate(
        [i.reshape(B, N, 1) for i in (i0, i1, i2, i3)], axis=-1)
    commit4 = stats[0, :Q]
    div4 = stats[1, :Q]
    all_losses = commit4 + div4
    mean_breakdown = jnp.mean(div4)
    mean_commit = jnp.mean(commit4)
    return (quantized, all_indices, all_losses, quantized, x,
            mean_breakdown, mean_commit)


# speedup vs baseline: 1.2411x; 1.2411x over previous
import functools

import jax
import jax.numpy as jnp
from jax.experimental import pallas as pl
from jax.experimental.pallas import tpu as pltpu

Q = 4
K = 1024
D = 256
TILE = 512


def _rownorm(m):
    sq = m * m
    a = sq[:, :128] + sq[:, 128:]
    at = jnp.swapaxes(a, 0, 1)
    s = at[0:8]
    for j in range(1, 16):
        s = s + at[8 * j:8 * (j + 1)]
    s = s[0:4] + s[4:8]
    s = s[0:2] + s[2:4]
    s = s[0:1] + s[1:2]
    return jnp.swapaxes(s, 0, 1)


def _rvq_body(x_ref, cbt2_ref, c123_ref,
              qout_ref, i0_ref, i1_ref, i2_ref, i3_ref, stats_ref,
              counts_ref, commit_ref, cn_ref):
    t = pl.program_id(0)
    nt = pl.num_programs(0)
    bn = nt * TILE

    @pl.when(t == 0)
    def _init():
        counts_ref[...] = jnp.zeros_like(counts_ref)
        commit_ref[...] = jnp.zeros_like(commit_ref)
        stats_ref[...] = jnp.zeros_like(stats_ref)
        for q in range(Q):
            et2 = cbt2_ref[q]
            cn_ref[q:q + 1, :] = 0.25 * jnp.sum(et2 * et2, axis=0,
                                                keepdims=True)

    idx_refs = (i0_ref, i1_ref, i2_ref, i3_ref)
    H = TILE // 2
    lane_iota = jax.lax.broadcasted_iota(jnp.int32, (H, K), 1)
    iota128 = jax.lax.broadcasted_iota(jnp.int32, (H, 128), 1)
    ones8 = jnp.ones((8, H), jnp.bfloat16)
    sls = (slice(0, H), slice(H, TILE))

    rs = [x_ref[sl, :] for sl in sls]
    qouts = [jnp.zeros_like(rs[0]) for _ in range(2)]
    for q in range(Q):
        rns = [_rownorm(r) for r in rs]
        scoress = [jax.lax.dot_general(
            r, cbt2_ref[q], (((1,), (0,)), ((), ())),
            preferred_element_type=jnp.float32) for r in rs]
        idx2s = []
        for h in range(2):
            rn, scores = rns[h], scoress[h]
            run_val = rn - scores[:, 0:128] + cn_ref[q:q + 1, 0:128]
            run_idx = iota128
            for c in range(1, K // 128):
                dv = (rn - scores[:, 128 * c:128 * (c + 1)]
                      + cn_ref[q:q + 1, 128 * c:128 * (c + 1)])
                upd = dv < run_val
                run_val = jnp.where(upd, dv, run_val)
                run_idx = jnp.where(upd, iota128 + 128 * c, run_idx)
            minv = jnp.min(run_val, axis=-1, keepdims=True)
            idx2s.append(jnp.min(
                jnp.where(run_val == minv, run_idx, K), axis=-1,
                keepdims=True))
        ohs = [(lane_iota == idx2).astype(jnp.bfloat16) for idx2 in idx2s]
        for h in range(2):
            idx_refs[q][sls[h], :] = idx2s[h]
        qh3s = [jax.lax.dot_general(
            oh, c123_ref[q], (((1,), (0,)), ((), ())),
            preferred_element_type=jnp.float32) for oh in ohs]
        cnt8s = [jax.lax.dot_general(
            ones8, oh, (((1,), (0,)), ((), ())),
            preferred_element_type=jnp.float32) for oh in ohs]
        commit_acc = []
        for h in range(2):
            qh3 = qh3s[h]
            qh = (qh3[:, :D] + qh3[:, D:2 * D]) + qh3[:, 2 * D:]
            diff = qh - rs[h]
            qz = rs[h] + diff
            commit_acc.append(jnp.sum(diff * diff, axis=0, keepdims=True))
            rs[h] = rs[h] - qz
            qouts[h] = qouts[h] + qz
        commit_ref[q:q + 1, :] = (commit_ref[q:q + 1, :]
                                  + (commit_acc[0] + commit_acc[1]))
        counts_ref[q:q + 1, :] = (counts_ref[q:q + 1, :]
                                  + (cnt8s[0][0:1, :] + cnt8s[1][0:1, :]))
    for h in range(2):
        qout_ref[sls[h], :] = qouts[h]

    @pl.when(t == nt - 1)
    def _epilogue():
        rows = jax.lax.broadcasted_iota(jnp.int32, (8, 128), 0)
        lanes = jax.lax.broadcasted_iota(jnp.int32, (8, 128), 1)
        stats = jnp.zeros((8, 128), jnp.float32)
        for q in range(Q):
            avg = counts_ref[q:q + 1, :] / float(bn)
            divq = jnp.sum(avg * jnp.log(avg + 1e-10))
            comq = jnp.sum(commit_ref[q:q + 1, :]) / float(bn * D)
            stats = (stats
                     + jnp.where((rows == 0) & (lanes == q), comq, 0.0)
                     + jnp.where((rows == 1) & (lanes == q), divq, 0.0))
        stats_ref[...] = stats


@functools.partial(jax.jit, static_argnames=("interpret",))
def kernel(x, codebooks, interpret=False):
    B, N, _ = x.shape
    bn = B * N
    nt = bn // TILE
    x2 = x.reshape(bn, D)
    cbt = jnp.swapaxes(codebooks, 1, 2)
    cbt2 = cbt + cbt
    mask = jnp.int32(-65536)
    v = jax.lax.bitcast_convert_type(codebooks, jnp.int32)
    h1 = jax.lax.bitcast_convert_type(v & mask, jnp.float32)
    r1 = codebooks - h1
    w = jax.lax.bitcast_convert_type(r1, jnp.int32)
    h2 = jax.lax.bitcast_convert_type(w & mask, jnp.float32)
    r2 = r1 - h2
    c123 = jnp.concatenate([h1.astype(jnp.bfloat16),
                            h2.astype(jnp.bfloat16),
                            r2.astype(jnp.bfloat16)], axis=2)

    outs = pl.pallas_call(
        _rvq_body,
        grid=(nt,),
        in_specs=[
            pl.BlockSpec((TILE, D), lambda t: (t, 0)),
            pl.BlockSpec((Q, D, K), lambda t: (0, 0, 0)),
            pl.BlockSpec((Q, K, 3 * D), lambda t: (0, 0, 0)),
        ],
        out_specs=[
            pl.BlockSpec((TILE, D), lambda t: (t, 0)),
            pl.BlockSpec((TILE, 1), lambda t: (t, 0)),
            pl.BlockSpec((TILE, 1), lambda t: (t, 0)),
            pl.BlockSpec((TILE, 1), lambda t: (t, 0)),
            pl.BlockSpec((TILE, 1), lambda t: (t, 0)),
            pl.BlockSpec((8, 128), lambda t: (0, 0)),
        ],
        out_shape=[
            jax.ShapeDtypeStruct((bn, D), jnp.float32),
            jax.ShapeDtypeStruct((bn, 1), jnp.int32),
            jax.ShapeDtypeStruct((bn, 1), jnp.int32),
            jax.ShapeDtypeStruct((bn, 1), jnp.int32),
            jax.ShapeDtypeStruct((bn, 1), jnp.int32),
            jax.ShapeDtypeStruct((8, 128), jnp.float32),
        ],
        scratch_shapes=[
            pltpu.VMEM((8, K), jnp.float32),
            pltpu.VMEM((8, D), jnp.float32),
            pltpu.VMEM((8, K), jnp.float32),
        ],
        compiler_params=pltpu.CompilerParams(
            dimension_semantics=("arbitrary",)),
        interpret=interpret,
    )(x2, cbt2, c123)

    qout, i0, i1, i2, i3, stats = outs
    quantized = qout.reshape(B, N, D)
    all_indices = jnp.concatenate(
        [i.reshape(B, N, 1) for i in (i0, i1, i2, i3)], axis=-1)
    commit4 = stats[0, :Q]
    div4 = stats[1, :Q]
    all_losses = commit4 + div4
    mean_breakdown = jnp.mean(div4)
    mean_commit = jnp.mean(commit4)
    return (quantized, all_indices, all_losses, quantized, x,
            mean_breakdown, mean_commit)


# speedup vs baseline: 1.3329x; 1.0739x over previous
import functools

import jax
import jax.numpy as jnp
from jax.experimental import pallas as pl
from jax.experimental.pallas import tpu as pltpu

Q = 4
K = 1024
D = 256
TILE = 512


def _rownorm(m):
    sq = m * m
    a = sq[:, :128] + sq[:, 128:]
    at = jnp.swapaxes(a, 0, 1)
    s = at[0:8]
    for j in range(1, 16):
        s = s + at[8 * j:8 * (j + 1)]
    s = s[0:4] + s[4:8]
    s = s[0:2] + s[2:4]
    s = s[0:1] + s[1:2]
    return jnp.swapaxes(s, 0, 1)


def _rvq_body(x_ref, cbt2_ref, cb_ref,
              qout_ref, i0_ref, i1_ref, i2_ref, i3_ref, stats_ref,
              counts_ref, commit_ref, cn_ref, c123_ref):
    t = pl.program_id(0)
    nt = pl.num_programs(0)
    bn = nt * TILE

    @pl.when(t == 0)
    def _init():
        counts_ref[...] = jnp.zeros_like(counts_ref)
        commit_ref[...] = jnp.zeros_like(commit_ref)
        stats_ref[...] = jnp.zeros_like(stats_ref)
        mask = jnp.int32(-65536)
        for q in range(Q):
            et2 = cbt2_ref[q]
            cn_ref[q:q + 1, :] = 0.25 * jnp.sum(et2 * et2, axis=0,
                                                keepdims=True)
            e = cb_ref[q]
            v = jax.lax.bitcast_convert_type(e, jnp.int32)
            h1 = jax.lax.bitcast_convert_type(v & mask, jnp.float32)
            r1 = e - h1
            w = jax.lax.bitcast_convert_type(r1, jnp.int32)
            h2 = jax.lax.bitcast_convert_type(w & mask, jnp.float32)
            r2 = r1 - h2
            c123_ref[q, :, 0:D] = h1.astype(jnp.bfloat16)
            c123_ref[q, :, D:2 * D] = h2.astype(jnp.bfloat16)
            c123_ref[q, :, 2 * D:] = r2.astype(jnp.bfloat16)

    idx_refs = (i0_ref, i1_ref, i2_ref, i3_ref)
    H = TILE // 2
    lane_iota = jax.lax.broadcasted_iota(jnp.int32, (H, K), 1)
    iota128 = jax.lax.broadcasted_iota(jnp.int32, (H, 128), 1)
    ones8 = jnp.ones((8, H), jnp.bfloat16)
    sls = (slice(0, H), slice(H, TILE))

    rs = [x_ref[sl, :] for sl in sls]
    qouts = [jnp.zeros_like(rs[0]) for _ in range(2)]
    for q in range(Q):
        rns = [_rownorm(r) for r in rs]
        scoress = [jax.lax.dot_general(
            r, cbt2_ref[q], (((1,), (0,)), ((), ())),
            preferred_element_type=jnp.float32) for r in rs]
        idx2s = []
        for h in range(2):
            rn, scores = rns[h], scoress[h]
            run_val = rn - scores[:, 0:128] + cn_ref[q:q + 1, 0:128]
            run_idx = iota128
            for c in range(1, K // 128):
                dv = (rn - scores[:, 128 * c:128 * (c + 1)]
                      + cn_ref[q:q + 1, 128 * c:128 * (c + 1)])
                upd = dv < run_val
                run_val = jnp.where(upd, dv, run_val)
                run_idx = jnp.where(upd, iota128 + 128 * c, run_idx)
            minv = jnp.min(run_val, axis=-1, keepdims=True)
            idx2s.append(jnp.min(
                jnp.where(run_val == minv, run_idx, K), axis=-1,
                keepdims=True))
        ohs = [(lane_iota == idx2).astype(jnp.bfloat16) for idx2 in idx2s]
        for h in range(2):
            idx_refs[q][sls[h], :] = idx2s[h]
        qh3s = [jax.lax.dot_general(
            oh, c123_ref[q], (((1,), (0,)), ((), ())),
            preferred_element_type=jnp.float32) for oh in ohs]

        cnt8s = [jax.lax.dot_general(
            ones8, oh, (((1,), (0,)), ((), ())),
            preferred_element_type=jnp.float32) for oh in ohs]
        commit_acc = []
        for h in range(2):
            qh3 = qh3s[h]
            qh = (qh3[:, :D] + qh3[:, D:2 * D]) + qh3[:, 2 * D:]
            diff = qh - rs[h]
            qz = rs[h] + diff
            commit_acc.append(jnp.sum(diff * diff, axis=0, keepdims=True))
            rs[h] = rs[h] - qz
            qouts[h] = qouts[h] + qz
        commit_ref[q:q + 1, :] = (commit_ref[q:q + 1, :]
                                  + (commit_acc[0] + commit_acc[1]))
        counts_ref[q:q + 1, :] = (counts_ref[q:q + 1, :]
                                  + (cnt8s[0][0:1, :] + cnt8s[1][0:1, :]))
    for h in range(2):
        qout_ref[sls[h], :] = qouts[h]

    @pl.when(t == nt - 1)
    def _epilogue():
        rows = jax.lax.broadcasted_iota(jnp.int32, (8, 128), 0)
        lanes = jax.lax.broadcasted_iota(jnp.int32, (8, 128), 1)
        stats = jnp.zeros((8, 128), jnp.float32)
        for q in range(Q):
            avg = counts_ref[q:q + 1, :] / float(bn)
            divq = jnp.sum(avg * jnp.log(avg + 1e-10))
            comq = jnp.sum(commit_ref[q:q + 1, :]) / float(bn * D)
            stats = (stats
                     + jnp.where((rows == 0) & (lanes == q), comq, 0.0)
                     + jnp.where((rows == 1) & (lanes == q), divq, 0.0))
        stats_ref[...] = stats


@functools.partial(jax.jit, static_argnames=("interpret",))
def kernel(x, codebooks, interpret=False):
    B, N, _ = x.shape
    bn = B * N
    nt = bn // TILE
    x2 = x.reshape(bn, D)
    cbt = jnp.swapaxes(codebooks, 1, 2)
    cbt2 = cbt + cbt

    outs = pl.pallas_call(
        _rvq_body,
        grid=(nt,),
        in_specs=[
            pl.BlockSpec((TILE, D), lambda t: (t, 0)),
            pl.BlockSpec((Q, D, K), lambda t: (0, 0, 0)),
            pl.BlockSpec((Q, K, D), lambda t: (0, 0, 0)),
        ],
        out_specs=[
            pl.BlockSpec((TILE, D), lambda t: (t, 0)),
            pl.BlockSpec((TILE, 1), lambda t: (t, 0)),
            pl.BlockSpec((TILE, 1), lambda t: (t, 0)),
            pl.BlockSpec((TILE, 1), lambda t: (t, 0)),
            pl.BlockSpec((TILE, 1), lambda t: (t, 0)),
            pl.BlockSpec((8, 128), lambda t: (0, 0)),
        ],
        out_shape=[
            jax.ShapeDtypeStruct((bn, D), jnp.float32),
            jax.ShapeDtypeStruct((bn, 1), jnp.int32),
            jax.ShapeDtypeStruct((bn, 1), jnp.int32),
            jax.ShapeDtypeStruct((bn, 1), jnp.int32),
            jax.ShapeDtypeStruct((bn, 1), jnp.int32),
            jax.ShapeDtypeStruct((8, 128), jnp.float32),
        ],
        scratch_shapes=[
            pltpu.VMEM((8, K), jnp.float32),
            pltpu.VMEM((8, D), jnp.float32),
            pltpu.VMEM((8, K), jnp.float32),
            pltpu.VMEM((Q, K, 3 * D), jnp.bfloat16),
        ],
        compiler_params=pltpu.CompilerParams(
            dimension_semantics=("arbitrary",)),
        interpret=interpret,
    )(x2, cbt2, codebooks)

    qout, i0, i1, i2, i3, stats = outs
    quantized = qout.reshape(B, N, D)
    all_indices = jnp.concatenate(
        [i.reshape(B, N, 1) for i in (i0, i1, i2, i3)], axis=-1)
    commit4 = stats[0, :Q]
    div4 = stats[1, :Q]
    all_losses = commit4 + div4
    mean_breakdown = jnp.mean(div4)
    mean_commit = jnp.mean(commit4)
    return (quantized, all_indices, all_losses, quantized, x,
            mean_breakdown, mean_commit)
